# baseline (speedup 1.0000x reference)
import jax
import jax.numpy as jnp
from jax.experimental import pallas as pl


def _vq_body(z_ref, w_ref, z2_ref, w2_ref, zq_ref, ind_ref, loss_ref):
    z = z_ref[...]
    w = w_ref[...]
    z2 = z2_ref[...]
    w2 = w2_ref[...]
    e = jax.lax.dot_general(z.astype(jnp.bfloat16), w.astype(jnp.bfloat16),
                            (((1,), (1,)), ((), ())),
                            preferred_element_type=jnp.float32)
    d = (z2 - 2.0 * e) + w2
    iota = jax.lax.broadcasted_iota(jnp.int32, d.shape, 1)
    m = jnp.min(d, axis=1, keepdims=True)
    ind = jnp.min(jnp.where(d == m, iota, d.shape[1]), axis=1)
    oh = (iota == ind[:, None]).astype(jnp.float32)
    zq = jax.lax.dot_general(oh, w, (((1,), (0,)), ((), ())),
                             preferred_element_type=jnp.float32)
    diff = zq - z
    zq_ref[...] = z + diff
    ind_ref[0, 0, :] = ind
    loss_ref[0, 0, :] = jnp.broadcast_to(jnp.sum(diff * diff), (128,))


def _vq_pallas(zf, W, z2, w2, tm, interpret=False):
    n, d_model = zf.shape
    k_cb = W.shape[0]
    g = n // tm
    out_shapes = (
        jax.ShapeDtypeStruct((n, d_model), jnp.float32),
        jax.ShapeDtypeStruct((g, 1, tm), jnp.int32),
        jax.ShapeDtypeStruct((g, 1, 128), jnp.float32),
    )
    return pl.pallas_call(
        _vq_body,
        grid=(g,),
        in_specs=[
            pl.BlockSpec((tm, d_model), lambda i: (i, 0)),
            pl.BlockSpec((k_cb, d_model), lambda i: (0, 0)),
            pl.BlockSpec((tm, 1), lambda i: (i, 0)),
            pl.BlockSpec((1, k_cb), lambda i: (0, 0)),
        ],
        out_specs=(
            pl.BlockSpec((tm, d_model), lambda i: (i, 0)),
            pl.BlockSpec((1, 1, tm), lambda i: (i, 0, 0)),
            pl.BlockSpec((1, 1, 128), lambda i: (i, 0, 0)),
        ),
        out_shape=out_shapes,
        interpret=interpret,
    )(zf, W, z2, w2)


def kernel(z, W):
    b, t, d_model = z.shape
    n = b * t
    tm = 2304
    zf = z.reshape(n, d_model)
    z2 = jnp.sum(z ** 2, axis=-1, keepdims=True).reshape(n, 1)
    w2 = jnp.sum(W ** 2, axis=-1).reshape(1, W.shape[0])
    zq_st, ind3, part = _vq_pallas(zf, W, z2, w2, tm)
    ind = ind3.reshape(b, t)
    loss = 2.0 * jnp.sum(part[:, 0, 0]) / (n * d_model)
    return zq_st.reshape(b, t, d_model), ind, loss


# speedup vs baseline: 1.3047x; 1.3047x over previous
import jax
import jax.numpy as jnp
from jax.experimental import pallas as pl


def _vq_body(z_ref, w_ref, w2_ref, zq_ref, ind_ref, loss_ref):
    z = z_ref[...]
    w = w_ref[...]
    w2 = w2_ref[...]
    z2 = jnp.sum(z * z, axis=1, keepdims=True)
    e = jax.lax.dot_general(z.astype(jnp.bfloat16), w.astype(jnp.bfloat16),
                            (((1,), (1,)), ((), ())),
                            preferred_element_type=jnp.float32)
    d = (z2 - 2.0 * e) + w2
    iota = jax.lax.broadcasted_iota(jnp.int32, d.shape, 1)
    m = jnp.min(d, axis=1, keepdims=True)
    ind = jnp.min(jnp.where(d == m, iota, d.shape[1]), axis=1)
    oh = (iota == ind[:, None]).astype(jnp.float32)
    zq = jax.lax.dot_general(oh, w, (((1,), (0,)), ((), ())),
                             preferred_element_type=jnp.float32)
    diff = zq - z
    zq_ref[...] = z + diff
    ind_ref[0, 0, :] = ind
    loss_ref[0, 0, :] = jnp.broadcast_to(jnp.sum(diff * diff), (128,))


def _vq_pallas(zf, W, w2, tm, interpret=False):
    n, d_model = zf.shape
    k_cb = W.shape[0]
    g = n // tm
    out_shapes = (
        jax.ShapeDtypeStruct((n, d_model), jnp.float32),
        jax.ShapeDtypeStruct((g, 1, tm), jnp.int32),
        jax.ShapeDtypeStruct((g, 1, 128), jnp.float32),
    )
    return pl.pallas_call(
        _vq_body,
        grid=(g,),
        in_specs=[
            pl.BlockSpec((tm, d_model), lambda i: (i, 0)),
            pl.BlockSpec((k_cb, d_model), lambda i: (0, 0)),
            pl.BlockSpec((1, k_cb), lambda i: (0, 0)),
        ],
        out_specs=(
            pl.BlockSpec((tm, d_model), lambda i: (i, 0)),
            pl.BlockSpec((1, 1, tm), lambda i: (i, 0, 0)),
            pl.BlockSpec((1, 1, 128), lambda i: (i, 0, 0)),
        ),
        out_shape=out_shapes,
        interpret=interpret,
    )(zf, W, w2)


def kernel(z, W):
    b, t, d_model = z.shape
    n = b * t
    tm = 1152
    zf = z.reshape(n, d_model)
    w2 = jnp.sum(W ** 2, axis=-1).reshape(1, W.shape[0])
    zq_st, ind3, part = _vq_pallas(zf, W, w2, tm)
    ind = ind3.reshape(b, t)
    loss = 2.0 * jnp.sum(part[:, 0, 0]) / (n * d_model)
    return zq_st.reshape(b, t, d_model), ind, loss


# speedup vs baseline: 1.3468x; 1.0322x over previous
import jax
import jax.numpy as jnp
from jax.experimental import pallas as pl


def _vq_body(z_ref, w_ref, w2_ref, zq_ref, ind_ref, loss_ref):
    z = z_ref[...]
    w = w_ref[...]
    w2 = w2_ref[...]
    z2 = jnp.sum(z * z, axis=1, keepdims=True)
    e = jax.lax.dot_general(z.astype(jnp.bfloat16), w.astype(jnp.bfloat16),
                            (((1,), (1,)), ((), ())),
                            preferred_element_type=jnp.float32)
    d = (z2 - 2.0 * e) + w2
    iota_f = jax.lax.broadcasted_iota(jnp.int32, d.shape, 1).astype(jnp.float32)
    m = jnp.min(d, axis=1, keepdims=True)
    ind_f = jnp.min(jnp.where(d == m, iota_f, float(d.shape[1])),
                    axis=1, keepdims=True)
    ind = ind_f[:, 0].astype(jnp.int32)
    oh = (iota_f == ind_f).astype(jnp.float32)
    zq = jax.lax.dot_general(oh, w, (((1,), (0,)), ((), ())),
                             preferred_element_type=jnp.float32)
    diff = zq - z
    zq_ref[...] = z + diff
    ind_ref[0, 0, :] = ind
    loss_ref[0, 0, :] = jnp.broadcast_to(jnp.sum(diff * diff), (128,))


def _vq_pallas(zf, W, w2, tm, interpret=False):
    n, d_model = zf.shape
    k_cb = W.shape[0]
    g = n // tm
    out_shapes = (
        jax.ShapeDtypeStruct((n, d_model), jnp.float32),
        jax.ShapeDtypeStruct((g, 1, tm), jnp.int32),
        jax.ShapeDtypeStruct((g, 1, 128), jnp.float32),
    )
    return pl.pallas_call(
        _vq_body,
        grid=(g,),
        in_specs=[
            pl.BlockSpec((tm, d_model), lambda i: (i, 0)),
            pl.BlockSpec((k_cb, d_model), lambda i: (0, 0)),
            pl.BlockSpec((1, k_cb), lambda i: (0, 0)),
        ],
        out_specs=(
            pl.BlockSpec((tm, d_model), lambda i: (i, 0)),
            pl.BlockSpec((1, 1, tm), lambda i: (i, 0, 0)),
            pl.BlockSpec((1, 1, 128), lambda i: (i, 0, 0)),
        ),
        out_shape=out_shapes,
        interpret=interpret,
    )(zf, W, w2)


def kernel(z, W):
    b, t, d_model = z.shape
    n = b * t
    tm = 1152
    zf = z.reshape(n, d_model)
    w2 = jnp.sum(W ** 2, axis=-1).reshape(1, W.shape[0])
    zq_st, ind3, part = _vq_pallas(zf, W, w2, tm)
    ind = ind3.reshape(b, t)
    loss = 2.0 * jnp.sum(part[:, 0, 0]) / (n * d_model)
    return zq_st.reshape(b, t, d_model), ind, loss
